# baseline (speedup 1.0000x reference)
import functools

import jax
import jax.numpy as jnp
from jax import lax
from jax.experimental import pallas as pl
from jax.experimental.pallas import tpu as pltpu
from jax.experimental.pallas import tpu_sc as plsc

VOCAB = 100000
EMB = 128
B = 4096
L = 200
H1 = 1024
H2 = 512
DOUT = 64

NC = 2
NS = 16
NW = NC * NS
LANE = 16
CHUNK = 100
NVEC = EMB // LANE

NBUF = 6


def _make_pool(nb):
    bpw = nb // NW
    nchunk = bpw * L // CHUNK
    ngroup = nchunk // NBUF

    def body(idx_hbm, emb_hbm, out_hbm, idx_v, rows_v, acc_v, *sems):
        wid = lax.axis_index("s") * NC + lax.axis_index("c")
        pltpu.sync_copy(idx_hbm.at[wid], idx_v)

        def start(c, k):
            pltpu.async_copy(emb_hbm.at[idx_v.at[c]], rows_v.at[k], sems[k])

        def wait(c, k):
            pltpu.make_async_copy(
                emb_hbm.at[idx_v.at[c]], rows_v.at[k], sems[k]
            ).wait()

        def accum(k, acc):
            buf = rows_v.at[k]

            def row(l, acc):
                return tuple(
                    acc[j] + buf[l, pl.ds(LANE * j, LANE)] for j in range(NVEC)
                )

            return lax.fori_loop(0, CHUNK, row, acc, unroll=5)

        for k in range(NBUF - 1):
            start(k, k)

        scale = jnp.float32(1.0 / L)

        def do_chunk(p, m, acc):
            c = NBUF * p + m
            nxt = c + NBUF - 1

            @pl.when(nxt < nchunk)
            def _():
                start(nxt, (m + NBUF - 1) % NBUF)

            wait(c, m)
            return accum(m, acc)

        def store(b, acc):
            for j in range(NVEC):
                acc_v[b, pl.ds(LANE * j, LANE)] = acc[j] * scale

        zeros = lambda: tuple(
            jnp.zeros((LANE,), jnp.float32) for _ in range(NVEC)
        )

        def group(p, carry):
            for r in range(NBUF // 2):
                acc = zeros()
                acc = do_chunk(p, 2 * r, acc)
                acc = do_chunk(p, 2 * r + 1, acc)
                store((NBUF // 2) * p + r, acc)
            return carry

        lax.fori_loop(0, ngroup, group, 0)
        for r in range((nchunk - NBUF * ngroup) // 2):
            acc = zeros()
            acc = do_chunk(ngroup, 2 * r, acc)
            acc = do_chunk(ngroup, 2 * r + 1, acc)
            store((NBUF // 2) * ngroup + r, acc)
        pltpu.sync_copy(acc_v, out_hbm.at[pl.ds(wid * bpw, bpw)])

    return pl.kernel(
        body,
        out_type=jax.ShapeDtypeStruct((nb, EMB), jnp.float32),
        mesh=plsc.VectorSubcoreMesh(core_axis_name="c", subcore_axis_name="s"),
        scratch_types=[
            pltpu.VMEM((nchunk, CHUNK), jnp.int32),
            pltpu.VMEM((NBUF, CHUNK, EMB), jnp.float32),
            pltpu.VMEM((bpw, EMB), jnp.float32),
        ] + [pltpu.SemaphoreType.DMA] * NBUF,
    )


NSPLIT = 2
BSPLIT = B // NSPLIT
_pool = _make_pool(BSPLIT)


MLP_BB = 512


def _mlp_body(x_ref, w1_ref, b1_ref, w2_ref, b2_ref, w3_ref, b3_ref, o_ref):
    dn = (((1,), (1,)), ((), ()))
    x = x_ref[...]
    h = lax.dot_general(x, w1_ref[...], dn, preferred_element_type=jnp.float32)
    h = jnp.maximum(h + b1_ref[...], 0.0)
    h = lax.dot_general(h, w2_ref[...], dn, preferred_element_type=jnp.float32)
    h = jnp.maximum(h + b2_ref[...], 0.0)
    h = lax.dot_general(h, w3_ref[...], dn, preferred_element_type=jnp.float32)
    o_ref[...] = h + b3_ref[...]


def _mlp(x, W1, b1, W2, b2, W3, b3):
    rep2 = lambda i: (0, 0)
    return pl.pallas_call(
        _mlp_body,
        grid=(BSPLIT // MLP_BB,),
        in_specs=[
            pl.BlockSpec((MLP_BB, EMB), lambda i: (i, 0)),
            pl.BlockSpec((H1, EMB), rep2),
            pl.BlockSpec((1, H1), rep2),
            pl.BlockSpec((H2, H1), rep2),
            pl.BlockSpec((1, H2), rep2),
            pl.BlockSpec((DOUT, H2), rep2),
            pl.BlockSpec((1, DOUT), rep2),
        ],
        out_specs=pl.BlockSpec((MLP_BB, DOUT), lambda i: (i, 0)),
        out_shape=jax.ShapeDtypeStruct((BSPLIT, DOUT), jnp.float32),
    )(x, W1, b1.reshape(1, H1), W2, b2.reshape(1, H2), W3, b3.reshape(1, DOUT))


def kernel(text, emb, W1, b1, W2, b2, W3, b3):
    nchunk = BSPLIT // NW * L // CHUNK
    idx = text.astype(jnp.int32).reshape(NSPLIT, NW, nchunk, CHUNK)
    pooled = [_pool(idx[i], emb) for i in range(NSPLIT)]
    outs = [_mlp(p, W1, b1, W2, b2, W3, b3) for p in pooled]
    return jnp.concatenate(outs, axis=0)


# speedup vs baseline: 1.0355x; 1.0355x over previous
import functools

import jax
import jax.numpy as jnp
from jax import lax
from jax.experimental import pallas as pl
from jax.experimental.pallas import tpu as pltpu
from jax.experimental.pallas import tpu_sc as plsc

VOCAB = 100000
EMB = 128
B = 4096
L = 200
H1 = 1024
H2 = 512
DOUT = 64

NC = 2
NS = 16
NW = NC * NS
LANE = 16
BPW = B // NW
CHUNK = 100
NCHUNK = BPW * L // CHUNK
NVEC = EMB // LANE

NBUF = 6
NGROUP = NCHUNK // NBUF


def _pool_body(idx_hbm, emb_hbm, out_hbm, idx_v, rows_v, acc_v, *sems):
    wid = lax.axis_index("s") * NC + lax.axis_index("c")
    pltpu.sync_copy(idx_hbm.at[wid], idx_v)

    def start(c, k):
        pltpu.async_copy(emb_hbm.at[idx_v.at[c]], rows_v.at[k], sems[k])

    def wait(c, k):
        pltpu.make_async_copy(
            emb_hbm.at[idx_v.at[c]], rows_v.at[k], sems[k]
        ).wait()

    def accum(k, acc):
        buf = rows_v.at[k]

        def row(l, acc):
            return tuple(
                acc[j] + buf[l, pl.ds(LANE * j, LANE)] for j in range(NVEC)
            )

        return lax.fori_loop(0, CHUNK, row, acc, unroll=10)

    for k in range(NBUF - 1):
        start(k, k)

    scale = jnp.float32(1.0 / L)

    def do_chunk(p, m, acc):
        c = NBUF * p + m
        nxt = c + NBUF - 1

        @pl.when(nxt < NCHUNK)
        def _():
            start(nxt, (m + NBUF - 1) % NBUF)

        wait(c, m)
        return accum(m, acc)

    def store(b, acc):
        for j in range(NVEC):
            acc_v[b, pl.ds(LANE * j, LANE)] = acc[j] * scale

    zeros = lambda: tuple(jnp.zeros((LANE,), jnp.float32) for _ in range(NVEC))

    def group(p, carry):
        for r in range(NBUF // 2):
            acc = zeros()
            acc = do_chunk(p, 2 * r, acc)
            acc = do_chunk(p, 2 * r + 1, acc)
            store((NBUF // 2) * p + r, acc)
        return carry

    lax.fori_loop(0, NGROUP, group, 0)
    for r in range((NCHUNK - NBUF * NGROUP) // 2):
        acc = zeros()
        acc = do_chunk(NGROUP, 2 * r, acc)
        acc = do_chunk(NGROUP, 2 * r + 1, acc)
        store((NBUF // 2) * NGROUP + r, acc)
    pltpu.sync_copy(acc_v, out_hbm.at[pl.ds(wid * BPW, BPW)])


@functools.partial(
    pl.kernel,
    out_type=jax.ShapeDtypeStruct((B, EMB), jnp.float32),
    mesh=plsc.VectorSubcoreMesh(core_axis_name="c", subcore_axis_name="s"),
    scratch_types=[
        pltpu.VMEM((NCHUNK, CHUNK), jnp.int32),
        pltpu.VMEM((NBUF, CHUNK, EMB), jnp.float32),
        pltpu.VMEM((BPW, EMB), jnp.float32),
    ] + [pltpu.SemaphoreType.DMA] * NBUF,
)
def _pool(idx_hbm, emb_hbm, out_hbm, idx_v, rows_v, acc_v, *sems):
    _pool_body(idx_hbm, emb_hbm, out_hbm, idx_v, rows_v, acc_v, *sems)


MLP_BB = 512


def _mlp_body(x_ref, w1_ref, b1_ref, w2_ref, b2_ref, w3_ref, b3_ref, o_ref):
    dn = (((1,), (1,)), ((), ()))
    x = x_ref[...]
    h = lax.dot_general(x, w1_ref[...], dn, preferred_element_type=jnp.float32)
    h = jnp.maximum(h + b1_ref[...], 0.0)
    h = lax.dot_general(h, w2_ref[...], dn, preferred_element_type=jnp.float32)
    h = jnp.maximum(h + b2_ref[...], 0.0)
    h = lax.dot_general(h, w3_ref[...], dn, preferred_element_type=jnp.float32)
    o_ref[...] = h + b3_ref[...]


def _mlp(x, W1, b1, W2, b2, W3, b3):
    rep2 = lambda i: (0, 0)
    return pl.pallas_call(
        _mlp_body,
        grid=(B // MLP_BB,),
        in_specs=[
            pl.BlockSpec((MLP_BB, EMB), lambda i: (i, 0)),
            pl.BlockSpec((H1, EMB), rep2),
            pl.BlockSpec((1, H1), rep2),
            pl.BlockSpec((H2, H1), rep2),
            pl.BlockSpec((1, H2), rep2),
            pl.BlockSpec((DOUT, H2), rep2),
            pl.BlockSpec((1, DOUT), rep2),
        ],
        out_specs=pl.BlockSpec((MLP_BB, DOUT), lambda i: (i, 0)),
        out_shape=jax.ShapeDtypeStruct((B, DOUT), jnp.float32),
    )(x, W1, b1.reshape(1, H1), W2, b2.reshape(1, H2), W3, b3.reshape(1, DOUT))


def kernel(text, emb, W1, b1, W2, b2, W3, b3):
    idx = text.astype(jnp.int32).reshape(NW, NCHUNK, CHUNK)
    pooled = _pool(idx, emb)
    return _mlp(pooled, W1, b1, W2, b2, W3, b3)


# speedup vs baseline: 1.0440x; 1.0082x over previous
import functools

import jax
import jax.numpy as jnp
from jax import lax
from jax.experimental import pallas as pl
from jax.experimental.pallas import tpu as pltpu
from jax.experimental.pallas import tpu_sc as plsc

VOCAB = 100000
EMB = 128
B = 4096
L = 200
H1 = 1024
H2 = 512
DOUT = 64

NC = 2
NS = 16
NW = NC * NS
LANE = 16
BPW = B // NW
CHUNK = 100
NCHUNK = BPW * L // CHUNK
NVEC = EMB // LANE

NBUF = 6
NGROUP = NCHUNK // NBUF


def _pool_body(idx_hbm, emb_hbm, out_hbm, idx_v, rows_v, acc_v, *sems):
    wid = lax.axis_index("s") * NC + lax.axis_index("c")
    pltpu.sync_copy(idx_hbm.at[wid], idx_v)

    def start(c, k):
        pltpu.async_copy(emb_hbm.at[idx_v.at[c]], rows_v.at[k], sems[k])

    def wait(c, k):
        pltpu.make_async_copy(
            emb_hbm.at[idx_v.at[c]], rows_v.at[k], sems[k]
        ).wait()

    def accum(k, acc):
        buf = rows_v.at[k]

        def row(l, acc):
            return tuple(
                acc[j] + buf[l, pl.ds(LANE * j, LANE)] for j in range(NVEC)
            )

        return lax.fori_loop(0, CHUNK, row, acc, unroll=10)

    for k in range(NBUF - 1):
        start(k, k)

    scale = jnp.float32(1.0 / L)

    def do_chunk(p, m, acc):
        c = NBUF * p + m
        nxt = c + NBUF - 1

        @pl.when(nxt < NCHUNK)
        def _():
            start(nxt, (m + NBUF - 1) % NBUF)

        wait(c, m)
        return accum(m, acc)

    def store(b, acc):
        for j in range(NVEC):
            acc_v[b, pl.ds(LANE * j, LANE)] = acc[j] * scale

    zeros = lambda: tuple(jnp.zeros((LANE,), jnp.float32) for _ in range(NVEC))

    def group(p, carry):
        for r in range(NBUF // 2):
            acc = zeros()
            acc = do_chunk(p, 2 * r, acc)
            acc = do_chunk(p, 2 * r + 1, acc)
            store((NBUF // 2) * p + r, acc)
        return carry

    lax.fori_loop(0, NGROUP, group, 0)
    for r in range((NCHUNK - NBUF * NGROUP) // 2):
        acc = zeros()
        acc = do_chunk(NGROUP, 2 * r, acc)
        acc = do_chunk(NGROUP, 2 * r + 1, acc)
        store((NBUF // 2) * NGROUP + r, acc)
    pltpu.sync_copy(acc_v, out_hbm.at[pl.ds(wid * BPW, BPW)])


@functools.partial(
    pl.kernel,
    out_type=jax.ShapeDtypeStruct((B, EMB), jnp.float32),
    mesh=plsc.VectorSubcoreMesh(core_axis_name="c", subcore_axis_name="s"),
    scratch_types=[
        pltpu.VMEM((NCHUNK, CHUNK), jnp.int32),
        pltpu.VMEM((NBUF, CHUNK, EMB), jnp.float32),
        pltpu.VMEM((BPW, EMB), jnp.float32),
    ] + [pltpu.SemaphoreType.DMA] * NBUF,
)
def _pool(idx_hbm, emb_hbm, out_hbm, idx_v, rows_v, acc_v, *sems):
    _pool_body(idx_hbm, emb_hbm, out_hbm, idx_v, rows_v, acc_v, *sems)


MLP_BB = 2048


def _mlp_body(x_ref, w1_ref, b1_ref, w2_ref, b2_ref, w3_ref, b3_ref, o_ref):
    dn = (((1,), (0,)), ((), ()))
    x = x_ref[...].astype(jnp.bfloat16)
    h = lax.dot_general(x, w1_ref[...], dn, preferred_element_type=jnp.float32)
    h = jnp.maximum(h + b1_ref[...], 0.0).astype(jnp.bfloat16)
    h = lax.dot_general(h, w2_ref[...], dn, preferred_element_type=jnp.float32)
    h = jnp.maximum(h + b2_ref[...], 0.0).astype(jnp.bfloat16)
    h = lax.dot_general(h, w3_ref[...], dn, preferred_element_type=jnp.float32)
    o_ref[...] = h + b3_ref[...]


def _mlp(x, W1, b1, W2, b2, W3, b3):
    rep2 = lambda i: (0, 0)
    bf = jnp.bfloat16
    return pl.pallas_call(
        _mlp_body,
        grid=(B // MLP_BB,),
        in_specs=[
            pl.BlockSpec((MLP_BB, EMB), lambda i: (i, 0)),
            pl.BlockSpec((EMB, H1), rep2),
            pl.BlockSpec((1, H1), rep2),
            pl.BlockSpec((H1, H2), rep2),
            pl.BlockSpec((1, H2), rep2),
            pl.BlockSpec((H2, DOUT), rep2),
            pl.BlockSpec((1, DOUT), rep2),
        ],
        out_specs=pl.BlockSpec((MLP_BB, DOUT), lambda i: (i, 0)),
        out_shape=jax.ShapeDtypeStruct((B, DOUT), jnp.float32),
    )(x, W1.T.astype(bf), b1.reshape(1, H1), W2.T.astype(bf),
      b2.reshape(1, H2), W3.T.astype(bf), b3.reshape(1, DOUT))


def kernel(text, emb, W1, b1, W2, b2, W3, b3):
    idx = text.astype(jnp.int32).reshape(NW, NCHUNK, CHUNK)
    pooled = _pool(idx, emb)
    return _mlp(pooled, W1, b1, W2, b2, W3, b3)
